# baseline (speedup 1.0000x reference)
import functools

import jax
import jax.numpy as jnp
from jax import lax
from jax.experimental import pallas as pl
from jax.experimental.pallas import tpu as pltpu
from jax.experimental.pallas import tpu_sc as plsc

_B = 4096
_D = 64

_NC = 2
_NS = 16
_NW = _NC * _NS
_PW = 2 * _B // _NW // 2


def _gather_body(tab_hbm, idx_hbm, o1_out, o2_out, ids_v, o1, o2, sem):
    wid = lax.axis_index("s") * _NC + lax.axis_index("c")
    poff = wid * _PW
    pltpu.sync_copy(idx_hbm.at[pl.ds(2 * poff, 2 * _PW)], ids_v)
    iota = lax.iota(jnp.int32, 16)

    def _group(g, carry):
        grp = ids_v[pl.ds(16 * g, 16)]
        for j in range(16):
            sid = jnp.sum(jnp.where(iota == j, grp, 0))
            row = 8 * g + j // 2
            o = o1 if j % 2 == 0 else o2
            pltpu.async_copy(tab_hbm.at[sid >> 3, sid & 7, :],
                             o.at[row, :], sem)
        return carry

    lax.fori_loop(0, 2 * _PW // 16, _group, 0)

    for buf in (o1, o2):
        pltpu.make_async_copy(o1_out.at[pl.ds(0, _PW)], buf, sem).wait()
    pltpu.sync_copy(o1, o1_out.at[pl.ds(poff, _PW)])
    pltpu.sync_copy(o2, o2_out.at[pl.ds(poff, _PW)])


_gather = functools.partial(
    pl.kernel,
    mesh=plsc.VectorSubcoreMesh(core_axis_name="c", subcore_axis_name="s"),
    compiler_params=pltpu.CompilerParams(needs_layout_passes=False),
    out_type=[jax.ShapeDtypeStruct((_B, _D), jnp.float32)] * 2,
    scratch_types=[
        pltpu.VMEM((2 * _PW,), jnp.int32),
        pltpu.VMEM((_PW, _D), jnp.float32),
        pltpu.VMEM((_PW, _D), jnp.float32),
        pltpu.SemaphoreType.DMA,
    ],
)(_gather_body)


def _box_body(m1_ref, m2_ref, d1_ref, d2_ref, tag_ref, ntag_ref, pred_ref):
    b1min = m1_ref[...]
    b2min = m2_ref[...]
    b1max = b1min + jnp.exp(d1_ref[...])
    b2max = b2min + jnp.exp(d2_ref[...])
    z = jnp.maximum(b1min, b2min)
    zz = jnp.minimum(b1max, b2max)
    lens = zz - z
    eps = jnp.finfo(jnp.float32).tiny
    tag = jnp.min(lens, axis=1) > 0.0
    safe_in = jnp.where(tag[:, None],
                        jnp.clip(jnp.maximum(lens, 0.0), eps, None), 1.0)
    overlap = jnp.sum(jnp.log(safe_in), axis=1)
    neg = jnp.maximum(-lens, 0.0)
    sq = jnp.sum(neg * neg, axis=1)
    disjoint = jnp.where(sq > 0.0,
                         jnp.sqrt(jnp.where(sq > 0.0, sq, 1.0)), 0.0)
    log_inter = jnp.where(tag, overlap, disjoint)
    lens2 = b2max - b2min
    log_box2 = jnp.sum(jnp.log(jnp.clip(jnp.maximum(lens2, 0.0), eps, None)),
                       axis=1)
    pos = jnp.where(tag, jnp.exp(log_inter - log_box2), log_inter)
    tag_i = tag.astype(jnp.int32)
    tag_ref[:, 0] = tag_i
    ntag_ref[:, 0] = 1 - tag_i
    pred_ref[:, 0] = 1.0 - pos
    pred_ref[:, 1] = pos


_box = pl.pallas_call(
    _box_body,
    out_shape=[
        jax.ShapeDtypeStruct((_B, 1), jnp.int32),
        jax.ShapeDtypeStruct((_B, 1), jnp.int32),
        jax.ShapeDtypeStruct((_B, 2), jnp.float32),
    ],
)


def kernel(ids, min_embedding, delta_embedding):
    ids_flat = ids.reshape(2 * _B).astype(jnp.int32)
    m1, m2 = _gather(min_embedding.reshape(12500, 8, _D), ids_flat)
    d1, d2 = _gather(delta_embedding.reshape(12500, 8, _D), ids_flat)
    tag_i, ntag_i, pred = _box(m1, m2, d1, d2)
    tag = tag_i.reshape(_B).astype(jnp.bool_)
    ntag = ntag_i.reshape(_B).astype(jnp.bool_)
    return tag, ntag, pred


# speedup vs baseline: 1.0696x; 1.0696x over previous
import functools

import jax
import jax.numpy as jnp
from jax import lax
from jax.experimental import pallas as pl
from jax.experimental.pallas import tpu as pltpu
from jax.experimental.pallas import tpu_sc as plsc

_B = 4096
_D = 64

_NC = 2
_NS = 16
_NW = _NC * _NS
_PW = 2 * _B // _NW // 2


def _gather_body(min_hbm, delta_hbm, idx_hbm,
                 m1_out, m2_out, d1_out, d2_out,
                 ids_v, om1, om2, od1, od2, sem):
    wid = lax.axis_index("s") * _NC + lax.axis_index("c")
    poff = wid * _PW
    pltpu.sync_copy(idx_hbm.at[pl.ds(2 * poff, 2 * _PW)], ids_v)
    iota = lax.iota(jnp.int32, 16)

    def _group(g, carry):
        grp = ids_v[pl.ds(16 * g, 16)]
        for j in range(16):
            sid = jnp.sum(jnp.where(iota == j, grp, 0))
            row = 8 * g + j // 2
            om, od = (om1, od1) if j % 2 == 0 else (om2, od2)
            pltpu.async_copy(min_hbm.at[sid >> 3, sid & 7, :],
                             om.at[row, :], sem)
            pltpu.async_copy(delta_hbm.at[sid >> 3, sid & 7, :],
                             od.at[row, :], sem)
        return carry

    lax.fori_loop(0, 2 * _PW // 16, _group, 0)

    for buf in (om1, om2, od1, od2):
        pltpu.make_async_copy(m1_out.at[pl.ds(0, _PW)], buf, sem).wait()
    pltpu.sync_copy(om1, m1_out.at[pl.ds(poff, _PW)])
    pltpu.sync_copy(om2, m2_out.at[pl.ds(poff, _PW)])
    pltpu.sync_copy(od1, d1_out.at[pl.ds(poff, _PW)])
    pltpu.sync_copy(od2, d2_out.at[pl.ds(poff, _PW)])


_gather = functools.partial(
    pl.kernel,
    mesh=plsc.VectorSubcoreMesh(core_axis_name="c", subcore_axis_name="s"),
    compiler_params=pltpu.CompilerParams(needs_layout_passes=False),
    out_type=[jax.ShapeDtypeStruct((_B, _D), jnp.float32)] * 4,
    scratch_types=[
        pltpu.VMEM((2 * _PW,), jnp.int32),
        pltpu.VMEM((_PW, _D), jnp.float32),
        pltpu.VMEM((_PW, _D), jnp.float32),
        pltpu.VMEM((_PW, _D), jnp.float32),
        pltpu.VMEM((_PW, _D), jnp.float32),
        pltpu.SemaphoreType.DMA,
    ],
)(_gather_body)


def _box_body(m1_ref, m2_ref, d1_ref, d2_ref, tag_ref, ntag_ref, pred_ref):
    b1min = m1_ref[...]
    b2min = m2_ref[...]
    b1max = b1min + jnp.exp(d1_ref[...])
    b2max = b2min + jnp.exp(d2_ref[...])
    z = jnp.maximum(b1min, b2min)
    zz = jnp.minimum(b1max, b2max)
    lens = zz - z
    eps = jnp.finfo(jnp.float32).tiny
    tag = jnp.min(lens, axis=1) > 0.0
    safe_in = jnp.where(tag[:, None],
                        jnp.clip(jnp.maximum(lens, 0.0), eps, None), 1.0)
    overlap = jnp.sum(jnp.log(safe_in), axis=1)
    neg = jnp.maximum(-lens, 0.0)
    sq = jnp.sum(neg * neg, axis=1)
    disjoint = jnp.where(sq > 0.0,
                         jnp.sqrt(jnp.where(sq > 0.0, sq, 1.0)), 0.0)
    log_inter = jnp.where(tag, overlap, disjoint)
    lens2 = b2max - b2min
    log_box2 = jnp.sum(jnp.log(jnp.clip(jnp.maximum(lens2, 0.0), eps, None)),
                       axis=1)
    pos = jnp.where(tag, jnp.exp(log_inter - log_box2), log_inter)
    tag_i = tag.astype(jnp.int32)
    tag_ref[:, 0] = tag_i
    ntag_ref[:, 0] = 1 - tag_i
    pred_ref[:, 0] = 1.0 - pos
    pred_ref[:, 1] = pos


_box = pl.pallas_call(
    _box_body,
    out_shape=[
        jax.ShapeDtypeStruct((_B, 1), jnp.int32),
        jax.ShapeDtypeStruct((_B, 1), jnp.int32),
        jax.ShapeDtypeStruct((_B, 2), jnp.float32),
    ],
)


def kernel(ids, min_embedding, delta_embedding):
    m1, m2, d1, d2 = _gather(min_embedding.reshape(12500, 8, _D),
                             delta_embedding.reshape(12500, 8, _D),
                             ids.reshape(2 * _B).astype(jnp.int32))
    tag_i, ntag_i, pred = _box(m1, m2, d1, d2)
    tag = tag_i.reshape(_B).astype(jnp.bool_)
    ntag = ntag_i.reshape(_B).astype(jnp.bool_)
    return tag, ntag, pred
